# baseline (speedup 1.0000x reference)
import functools

import jax
import jax.numpy as jnp
from jax import lax
from jax.experimental import pallas as pl
from jax.experimental.pallas import tpu as pltpu
from jax.experimental.pallas import tpu_sc as plsc

NUM_USER = 100000
NUM_ITEM = 100000
EMBED = 64
B = 4096
WEIGHT_DECAY = 0.0001

NC = 2
NS = 16
NW = NC * NS
BPW = B // NW


def _sc_gather3_body(user_h, pos_h, neg_h, utab_h, itab_h,
                     ue_h, pe_h, ne_h,
                     idx_u, idx_p, idx_n, sem):
    wid = lax.axis_index("s") * NC + lax.axis_index("c")
    base = wid * BPW
    pltpu.sync_copy(user_h.at[pl.ds(base, BPW)], idx_u)
    pltpu.sync_copy(pos_h.at[pl.ds(base, BPW)], idx_p)
    pltpu.sync_copy(neg_h.at[pl.ds(base, BPW)], idx_n)

    def issue(c, _):
        k0 = c * 16
        vu = idx_u[pl.ds(k0, 16)]
        vp = idx_p[pl.ds(k0, 16)]
        vn = idx_n[pl.ds(k0, 16)]
        for j in range(16):
            pltpu.async_copy(utab_h.at[vu[j]], ue_h.at[base + k0 + j], sem)
            pltpu.async_copy(itab_h.at[vp[j]], pe_h.at[base + k0 + j], sem)
            pltpu.async_copy(itab_h.at[vn[j]], ne_h.at[base + k0 + j], sem)
        return _

    lax.fori_loop(0, BPW // 16, issue, None)
    for out_h in (ue_h, pe_h, ne_h):
        pltpu.make_async_copy(
            utab_h.at[pl.ds(0, BPW)], out_h.at[pl.ds(base, BPW)], sem
        ).wait()


@functools.cache
def _sc_gather3():
    return pl.kernel(
        _sc_gather3_body,
        out_type=[jax.ShapeDtypeStruct((B, EMBED), jnp.float32)] * 3,
        mesh=plsc.VectorSubcoreMesh(core_axis_name="c", subcore_axis_name="s",
                                    num_cores=NC, num_subcores=NS),
        scratch_types=[
            pltpu.VMEM((BPW,), jnp.int32),
            pltpu.VMEM((BPW,), jnp.int32),
            pltpu.VMEM((BPW,), jnp.int32),
            pltpu.SemaphoreType.DMA,
        ],
    )


def _tc_scalars_body(ue_ref, pe_ref, ne_ref, loss_ref, nlp_ref, reg_ref):
    ue = ue_ref[...]
    pe = pe_ref[...]
    ne = ne_ref[...]
    pos_out = jnp.sum(ue * pe, axis=1, keepdims=True)
    neg_out = jnp.sum(ue * ne, axis=1, keepdims=True)
    out = pos_out - neg_out
    log_prob = jnp.sum(jax.nn.log_sigmoid(out))
    reg = WEIGHT_DECAY * (jnp.sum(ue * ue) + jnp.sum(pe * pe)
                          + jnp.sum(ne * ne))
    nlp_ref[0, 0] = -log_prob
    reg_ref[0, 0] = reg
    loss_ref[0, 0] = -log_prob + reg


def _tc_scalars(ue, pe, ne):
    return pl.pallas_call(
        _tc_scalars_body,
        out_shape=[jax.ShapeDtypeStruct((1, 1), jnp.float32)] * 3,
        out_specs=[pl.BlockSpec(memory_space=pltpu.SMEM)] * 3,
    )(ue, pe, ne)


def kernel(user, pos, neg, history, history_mask, user_table, item_table):
    ue, pe, ne = _sc_gather3()(user, pos, neg, user_table, item_table)
    loss, nlp, reg = _tc_scalars(ue, pe, ne)
    return (loss[0, 0], nlp[0, 0], reg[0, 0], ue, pe, ne)


# speedup vs baseline: 3.3855x; 3.3855x over previous
import functools

import jax
import jax.numpy as jnp
from jax import lax
from jax.experimental import pallas as pl
from jax.experimental.pallas import tpu as pltpu
from jax.experimental.pallas import tpu_sc as plsc

NUM_USER = 100000
NUM_ITEM = 100000
EMBED = 64
B = 4096
WEIGHT_DECAY = 0.0001

NC = 2
NS = 16
NW = NC * NS
BPW = B // NW
NCHUNK = BPW // 16

HALF = 50048
TW = 2176
TSTEPS = HALF // TW


def _transpose_body(in1_ref, in2_ref, out_ref):
    cat = jnp.concatenate([in1_ref[...], in2_ref[...]], axis=0)
    out_ref[...] = jnp.transpose(cat, (1, 0))


def _pack_table(tab_t):
    return pl.pallas_call(
        _transpose_body,
        grid=(TSTEPS,),
        in_specs=[
            pl.BlockSpec((EMBED, TW), lambda c: (0, c)),
            pl.BlockSpec((EMBED, TW), lambda c: (0, c + TSTEPS)),
        ],
        out_specs=pl.BlockSpec((TW, 2 * EMBED), lambda c: (c, 0)),
        out_shape=jax.ShapeDtypeStruct((HALF, 2 * EMBED), jnp.float32),
    )(tab_t, tab_t)


def _sc_gather3_body(user_h, pos_h, neg_h, su_h, si_h,
                     bu_h, bp_h, bn_h,
                     idx_u, idx_p, idx_n, rows_u, rows_p, rows_n,
                     sem_u, sem_p, sem_n):
    wid = lax.axis_index("s") * NC + lax.axis_index("c")
    base = wid * BPW

    pltpu.sync_copy(user_h.at[pl.ds(base, BPW)], idx_u)
    pltpu.sync_copy(pos_h.at[pl.ds(base, BPW)], idx_p)
    pltpu.sync_copy(neg_h.at[pl.ds(base, BPW)], idx_n)

    def fold(c, _):
        k0 = c * 16
        for ref in (idx_u, idx_p, idx_n):
            v = ref[pl.ds(k0, 16)]
            ref[pl.ds(k0, 16)] = jnp.where(v >= HALF, v - HALF, v)
        return _

    lax.fori_loop(0, NCHUNK, fold, None)

    cu = pltpu.async_copy(su_h.at[idx_u], rows_u, sem_u)
    cp = pltpu.async_copy(si_h.at[idx_p], rows_p, sem_p)
    cn = pltpu.async_copy(si_h.at[idx_n], rows_n, sem_n)
    for copy, rows_v, out_h in ((cu, rows_u, bu_h),
                                (cp, rows_p, bp_h),
                                (cn, rows_n, bn_h)):
        copy.wait()
        pltpu.sync_copy(rows_v, out_h.at[pl.ds(base, BPW)])


@functools.cache
def _sc_gather3():
    return pl.kernel(
        _sc_gather3_body,
        out_type=[jax.ShapeDtypeStruct((B, 2 * EMBED), jnp.float32)] * 3,
        mesh=plsc.VectorSubcoreMesh(core_axis_name="c", subcore_axis_name="s",
                                    num_cores=NC, num_subcores=NS),
        scratch_types=[
            pltpu.VMEM((BPW,), jnp.int32),
            pltpu.VMEM((BPW,), jnp.int32),
            pltpu.VMEM((BPW,), jnp.int32),
            pltpu.VMEM((BPW, 2 * EMBED), jnp.float32),
            pltpu.VMEM((BPW, 2 * EMBED), jnp.float32),
            pltpu.VMEM((BPW, 2 * EMBED), jnp.float32),
            pltpu.SemaphoreType.DMA,
            pltpu.SemaphoreType.DMA,
            pltpu.SemaphoreType.DMA,
        ],
    )


def _final_body(user_ref, pos_ref, neg_ref, bu_ref, bp_ref, bn_ref,
                ue_ref, pe_ref, ne_ref, loss_ref, nlp_ref, reg_ref):
    def half(idx_1d, buf):
        h = (idx_1d[...] >= HALF).reshape(1, B)
        t = jnp.transpose(buf[...], (1, 0))
        return jnp.where(h, t[EMBED:, :], t[:EMBED, :])

    ue = half(user_ref, bu_ref)
    pe = half(pos_ref, bp_ref)
    ne = half(neg_ref, bn_ref)
    ue_ref[...] = ue
    pe_ref[...] = pe
    ne_ref[...] = ne
    pos_out = jnp.sum(ue * pe, axis=0, keepdims=True)
    neg_out = jnp.sum(ue * ne, axis=0, keepdims=True)
    out = pos_out - neg_out
    log_prob = jnp.sum(jax.nn.log_sigmoid(out))
    reg = WEIGHT_DECAY * (jnp.sum(ue * ue) + jnp.sum(pe * pe)
                          + jnp.sum(ne * ne))
    nlp_ref[0, 0] = -log_prob
    reg_ref[0, 0] = reg
    loss_ref[0, 0] = -log_prob + reg


def _tc_final(user, pos, neg, bu, bp, bn):
    return pl.pallas_call(
        _final_body,
        out_shape=[jax.ShapeDtypeStruct((EMBED, B), jnp.float32)] * 3
        + [jax.ShapeDtypeStruct((1, 1), jnp.float32)] * 3,
        out_specs=[pl.BlockSpec((EMBED, B), lambda: (0, 0))] * 3
        + [pl.BlockSpec(memory_space=pltpu.SMEM)] * 3,
    )(user, pos, neg, bu, bp, bn)


def kernel(user, pos, neg, history, history_mask, user_table, item_table):
    su = _pack_table(user_table.T)
    si = _pack_table(item_table.T)
    bu, bp, bn = _sc_gather3()(user, pos, neg, su, si)
    uet, pet, net, loss, nlp, reg = _tc_final(user, pos, neg, bu, bp, bn)
    return (loss[0, 0], nlp[0, 0], reg[0, 0], uet.T, pet.T, net.T)
